# baseline (speedup 1.0000x reference)
import jax
import jax.numpy as jnp
from jax import lax
from jax.experimental import pallas as pl
from jax.experimental.pallas import tpu as pltpu


def _make_body(H, Tc, B, n_chunks, unroll):
    G = 4 * H

    def gate_math(g, c):
        sig = jax.nn.sigmoid(g[:, :3 * H])
        gg = jnp.tanh(g[:, 3 * H:])
        c_new = sig[:, H:2 * H] * c + sig[:, :H] * gg
        h_new = sig[:, 2 * H:3 * H] * jnp.tanh(c_new)
        return h_new, c_new

    def body(x_ref, wih0_ref, wih1_ref, whh0_ref, whh1_ref, b_ref,
             h_ref, c_ref, xbuf, g0buf, g1buf):
        c_idx = pl.program_id(0)

        w0 = whh0_ref[...]
        w1 = whh1_ref[...]

        def hpart(h, w):
            return jnp.dot(h.astype(jnp.bfloat16), w,
                           preferred_element_type=jnp.float32)

        def step_l0(s, carry):
            _, c0, p0 = carry
            row = pl.multiple_of(s * B, B)
            g0 = g0buf[pl.ds(row, B), :] + p0
            h0n, c0n = gate_math(g0, c0)
            xbuf[pl.ds(row, B), :] = h0n.astype(jnp.bfloat16)
            return h0n, c0n, hpart(h0n, w0)

        def step_fused(s, carry):
            _, c0, _, c1, p0, p1 = carry
            row = pl.multiple_of(s * B, B)
            g0 = g0buf[pl.ds(row, B), :] + p0
            h0n, c0n = gate_math(g0, c0)
            xbuf[pl.ds(row, B), :] = h0n.astype(jnp.bfloat16)
            g1 = g1buf[pl.ds(row, B), :] + p1
            h1n, c1n = gate_math(g1, c1)
            return h0n, c0n, h1n, c1n, hpart(h0n, w0), hpart(h1n, w1)

        def step_l1(s, carry):
            _, c1, p1 = carry
            row = pl.multiple_of(s * B, B)
            g1 = g1buf[pl.ds(row, B), :] + p1
            h1n, c1n = gate_math(g1, c1)
            return h1n, c1n, hpart(h1n, w1)

        xt = jnp.transpose(x_ref[...], (1, 0, 2)).reshape(Tc * B, H)
        g0buf[...] = jnp.dot(xt.astype(jnp.bfloat16), wih0_ref[...],
                             preferred_element_type=jnp.float32) + b_ref[0]

        @pl.when(c_idx == 0)
        def _():
            z = jnp.zeros((B, H), jnp.float32)
            zg = jnp.zeros((B, G), jnp.float32)
            h0, c0, _ = lax.fori_loop(0, Tc, step_l0, (z, z, zg),
                                      unroll=unroll)
            h_ref[0], c_ref[0] = h0, c0
            h_ref[1] = jnp.zeros((B, H), jnp.float32)
            c_ref[1] = jnp.zeros((B, H), jnp.float32)

        @pl.when(c_idx > 0)
        def _():
            g1buf[...] = jnp.dot(xbuf[...], wih1_ref[...],
                                 preferred_element_type=jnp.float32) + b_ref[1]
            h0, c0 = h_ref[0], c_ref[0]
            h1, c1 = h_ref[1], c_ref[1]
            carry = (h0, c0, h1, c1, hpart(h0, w0), hpart(h1, w1))
            h0, c0, h1, c1, _, _ = lax.fori_loop(0, Tc, step_fused, carry,
                                                 unroll=unroll)
            h_ref[0], c_ref[0] = h0, c0
            h_ref[1], c_ref[1] = h1, c1

        @pl.when(c_idx == n_chunks - 1)
        def _():
            g1buf[...] = jnp.dot(xbuf[...], wih1_ref[...],
                                 preferred_element_type=jnp.float32) + b_ref[1]
            h1, c1 = h_ref[1], c_ref[1]
            carry = (h1, c1, hpart(h1, w1))
            h1, c1, _ = lax.fori_loop(0, Tc, step_l1, carry, unroll=unroll)
            h_ref[1], c_ref[1] = h1, c1

    return body


def kernel(in_seq, w_ih0, w_ihr, w_hh, b):
    B, T, D = in_seq.shape
    L, H, G = w_hh.shape
    assert L == 2 and G == 4 * H and B % 8 == 0
    Tc = 16 if T % 16 == 0 else T
    n_chunks = T // Tc

    x = in_seq
    wih0 = w_ih0.astype(jnp.bfloat16)
    wih1 = w_ihr[0].astype(jnp.bfloat16)
    whh0 = w_hh[0].astype(jnp.bfloat16)
    whh1 = w_hh[1].astype(jnp.bfloat16)

    body = _make_body(H, Tc, B, n_chunks, unroll=16)

    out_shapes = (
        jax.ShapeDtypeStruct((L, B, H), jnp.float32),
        jax.ShapeDtypeStruct((L, B, H), jnp.float32),
    )
    h_out, c_out = pl.pallas_call(
        body,
        out_shape=out_shapes,
        grid=(n_chunks,),
        in_specs=[
            pl.BlockSpec((B, Tc, D), lambda c: (0, c, 0)),
            pl.BlockSpec((D, G), lambda c: (0, 0)),
            pl.BlockSpec((H, G), lambda c: (0, 0)),
            pl.BlockSpec((H, G), lambda c: (0, 0)),
            pl.BlockSpec((H, G), lambda c: (0, 0)),
            pl.BlockSpec((L, 1, G), lambda c: (0, 0, 0)),
        ],
        out_specs=(
            pl.BlockSpec((L, B, H), lambda c: (0, 0, 0)),
            pl.BlockSpec((L, B, H), lambda c: (0, 0, 0)),
        ),
        scratch_shapes=[
            pltpu.VMEM((Tc * B, H), jnp.bfloat16),
            pltpu.VMEM((Tc * B, G), jnp.float32),
            pltpu.VMEM((Tc * B, G), jnp.float32),
        ],
        compiler_params=pltpu.CompilerParams(
            dimension_semantics=("arbitrary",),
            allow_input_fusion=[False, True, True, True, True, True],
            vmem_limit_bytes=48 * 2 ** 20),
    )(x, wih0, wih1, whh0, whh1, b)

    return h_out, c_out


# speedup vs baseline: 1.0495x; 1.0495x over previous
import jax
import jax.numpy as jnp
from jax import lax
from jax.experimental import pallas as pl
from jax.experimental.pallas import tpu as pltpu


def _make_body(H, Tc, B, n_chunks, unroll):
    G = 4 * H

    def gate_math(g, c):
        sig = jax.nn.sigmoid(g[:, :3 * H])
        gg = jnp.tanh(g[:, 3 * H:])
        c_new = sig[:, H:2 * H] * c + sig[:, :H] * gg
        h_new = sig[:, 2 * H:3 * H] * jnp.tanh(c_new)
        return h_new, c_new

    def body(x_ref, wih0_ref, wih1_ref, whh0_ref, whh1_ref, b_ref,
             h_ref, c_ref, xbuf, g0buf, g1buf):
        c_idx = pl.program_id(0)

        w0 = whh0_ref[...]
        w1 = whh1_ref[...]

        def hpart(h, w):
            return jnp.dot(h.astype(jnp.bfloat16), w,
                           preferred_element_type=jnp.float32)

        def step_l0(s, carry):
            _, c0, p0 = carry
            row = pl.multiple_of(s * B, B)
            g0 = g0buf[pl.ds(row, B), :] + p0
            h0n, c0n = gate_math(g0, c0)
            xbuf[pl.ds(row, B), :] = h0n.astype(jnp.bfloat16)
            return h0n, c0n, hpart(h0n, w0)

        def step_fused(s, carry):
            _, c0, _, c1, p0, p1 = carry
            row = pl.multiple_of(s * B, B)
            g0 = g0buf[pl.ds(row, B), :] + p0
            h0n, c0n = gate_math(g0, c0)
            xbuf[pl.ds(row, B), :] = h0n.astype(jnp.bfloat16)
            g1 = g1buf[pl.ds(row, B), :] + p1
            h1n, c1n = gate_math(g1, c1)
            return h0n, c0n, h1n, c1n, hpart(h0n, w0), hpart(h1n, w1)

        def step_l1(s, carry):
            _, c1, p1 = carry
            row = pl.multiple_of(s * B, B)
            g1 = g1buf[pl.ds(row, B), :] + p1
            h1n, c1n = gate_math(g1, c1)
            return h1n, c1n, hpart(h1n, w1)

        xt = jnp.transpose(x_ref[...], (1, 0, 2)).reshape(Tc * B, H)
        g0buf[...] = jnp.dot(xt.astype(jnp.bfloat16), wih0_ref[...],
                             preferred_element_type=jnp.float32) + b_ref[0]

        @pl.when(c_idx == 0)
        def _():
            z = jnp.zeros((B, H), jnp.float32)
            zg = jnp.zeros((B, G), jnp.float32)
            h0, c0, _ = lax.fori_loop(0, Tc, step_l0, (z, z, zg),
                                      unroll=unroll)
            h_ref[0], c_ref[0] = h0, c0
            h_ref[1] = jnp.zeros((B, H), jnp.float32)
            c_ref[1] = jnp.zeros((B, H), jnp.float32)

        @pl.when(c_idx > 0)
        def _():
            g1buf[...] = jnp.dot(xbuf[...], wih1_ref[...],
                                 preferred_element_type=jnp.float32) + b_ref[1]
            h0, c0 = h_ref[0], c_ref[0]
            h1, c1 = h_ref[1], c_ref[1]
            carry = (h0, c0, h1, c1, hpart(h0, w0), hpart(h1, w1))
            h0, c0, h1, c1, _, _ = lax.fori_loop(0, Tc, step_fused, carry,
                                                 unroll=unroll)
            h_ref[0], c_ref[0] = h0, c0
            h_ref[1], c_ref[1] = h1, c1

        @pl.when(c_idx == n_chunks - 1)
        def _():
            g1buf[...] = jnp.dot(xbuf[...], wih1_ref[...],
                                 preferred_element_type=jnp.float32) + b_ref[1]
            h1, c1 = h_ref[1], c_ref[1]
            carry = (h1, c1, hpart(h1, w1))
            h1, c1, _ = lax.fori_loop(0, Tc, step_l1, carry, unroll=unroll)
            h_ref[1], c_ref[1] = h1, c1

    return body


def kernel(in_seq, w_ih0, w_ihr, w_hh, b):
    B, T, D = in_seq.shape
    L, H, G = w_hh.shape
    assert L == 2 and G == 4 * H and B % 8 == 0
    Tc = 64 if T % 64 == 0 else T
    n_chunks = T // Tc

    x = in_seq
    wih0 = w_ih0.astype(jnp.bfloat16)
    wih1 = w_ihr[0].astype(jnp.bfloat16)
    whh0 = w_hh[0].astype(jnp.bfloat16)
    whh1 = w_hh[1].astype(jnp.bfloat16)

    body = _make_body(H, Tc, B, n_chunks, unroll=64)

    out_shapes = (
        jax.ShapeDtypeStruct((L, B, H), jnp.float32),
        jax.ShapeDtypeStruct((L, B, H), jnp.float32),
    )
    h_out, c_out = pl.pallas_call(
        body,
        out_shape=out_shapes,
        grid=(n_chunks,),
        in_specs=[
            pl.BlockSpec((B, Tc, D), lambda c: (0, c, 0)),
            pl.BlockSpec((D, G), lambda c: (0, 0)),
            pl.BlockSpec((H, G), lambda c: (0, 0)),
            pl.BlockSpec((H, G), lambda c: (0, 0)),
            pl.BlockSpec((H, G), lambda c: (0, 0)),
            pl.BlockSpec((L, 1, G), lambda c: (0, 0, 0)),
        ],
        out_specs=(
            pl.BlockSpec((L, B, H), lambda c: (0, 0, 0)),
            pl.BlockSpec((L, B, H), lambda c: (0, 0, 0)),
        ),
        scratch_shapes=[
            pltpu.VMEM((Tc * B, H), jnp.bfloat16),
            pltpu.VMEM((Tc * B, G), jnp.float32),
            pltpu.VMEM((Tc * B, G), jnp.float32),
        ],
        compiler_params=pltpu.CompilerParams(
            dimension_semantics=("arbitrary",),
            allow_input_fusion=[False, True, True, True, True, True],
            vmem_limit_bytes=48 * 2 ** 20),
    )(x, wih0, wih1, whh0, whh1, b)

    return h_out, c_out
